# Initial kernel scaffold; baseline (speedup 1.0000x reference)
import jax
import jax.numpy as jnp
from jax.experimental import pallas as pl


def kernel(x, timestamps, table):
    raise NotImplementedError("write your pallas kernel here")


# speedup vs baseline: 8.1681x; 8.1681x over previous
import functools

import jax
import jax.numpy as jnp
from jax import lax
from jax.experimental import pallas as pl
from jax.experimental.pallas import tpu as pltpu
from jax.experimental.pallas import tpu_sc as plsc

VOCAB_ = 100000
D_ = 128
B_ = 4096
S_ = 200
TOTAL_ = B_ * S_

NC_ = 2
NS_ = 16
NW_ = NC_ * NS_
PER_W_ = TOTAL_ // NW_
CHUNK_ = 512
N_CHUNKS_ = PER_W_ // CHUNK_


def _embed_body(x_hbm, table_hbm, out_hbm, idx_v, rows_v, gsem):
    wid = lax.axis_index("s") * NC_ + lax.axis_index("c")
    base = wid * PER_W_

    def chunk(g, carry):
        off = pl.multiple_of(base + g * CHUNK_, CHUNK_)
        pltpu.sync_copy(x_hbm.at[pl.ds(off, CHUNK_)], idx_v)
        pltpu.async_copy(table_hbm.at[idx_v], rows_v, gsem).wait()
        pltpu.sync_copy(rows_v, out_hbm.at[pl.ds(off, CHUNK_)])
        return carry

    lax.fori_loop(0, N_CHUNKS_, chunk, 0)


@jax.jit
def _embed(x_flat, table):
    mesh = plsc.VectorSubcoreMesh(core_axis_name="c", subcore_axis_name="s")
    run = pl.kernel(
        _embed_body,
        out_type=jax.ShapeDtypeStruct((TOTAL_, D_), jnp.float32),
        mesh=mesh,
        scratch_types=[
            pltpu.VMEM((CHUNK_,), jnp.int32),
            pltpu.VMEM((CHUNK_, D_), jnp.float32),
            pltpu.SemaphoreType.DMA,
        ],
    )
    return run(x_flat, table)


def kernel(x, timestamps, table):
    del timestamps
    x_flat = jnp.reshape(x, (TOTAL_,)).astype(jnp.int32)
    out = _embed(x_flat, table)
    return jnp.reshape(out, (B_, S_, D_))
